# Initial kernel scaffold; baseline (speedup 1.0000x reference)
import jax
import jax.numpy as jnp
from jax.experimental import pallas as pl


def kernel(node_features, edge_triples, num_nodes, W_in, b_in, basis0, att0, rootW0, rootb0, ln_s0, ln_b0, basis1, att1, rootW1, rootb1, ln_s1, ln_b1):
    raise NotImplementedError("write your pallas kernel here")


# speedup vs baseline: 17.5748x; 17.5748x over previous
import functools

import jax
import jax.numpy as jnp
from jax import lax
from jax.experimental import pallas as pl
from jax.experimental.pallas import tpu as pltpu
from jax.experimental.pallas import tpu_sc as plsc

N = 10000
E = 320000
R = 8
H = 64

NC = 2
NS = 16
NW = NC * NS
EPW = E // NW
S = 80
CH = EPW // S
N_PAD = 10240
RPT = N_PAD // NS
ZR = 128

RB = 2000


def _enc_body(nf, win, bin_, wcat, x_out, z_out):
    x = jnp.dot(nf[...], win[...], preferred_element_type=jnp.float32) + bin_[...]
    x_out[...] = x
    z_out[...] = jnp.dot(x, wcat[...], preferred_element_type=jnp.float32)


def _encode(nf, W_in, b_in, Wcat0):
    return pl.pallas_call(
        _enc_body,
        grid=(N // RB,),
        in_specs=[
            pl.BlockSpec((RB, 128), lambda i: (i, 0)),
            pl.BlockSpec((128, H), lambda i: (0, 0)),
            pl.BlockSpec((1, H), lambda i: (0, 0)),
            pl.BlockSpec((H, R * H), lambda i: (0, 0)),
        ],
        out_specs=[
            pl.BlockSpec((RB, H), lambda i: (i, 0)),
            pl.BlockSpec((RB, R * H), lambda i: (i, 0)),
        ],
        out_shape=[
            jax.ShapeDtypeStruct((N, H), jnp.float32),
            jax.ShapeDtypeStruct((N, R * H), jnp.float32),
        ],
    )(nf, W_in, b_in, Wcat0)


def _layer_tail(h, s_ref, b_ref):
    mu = jnp.mean(h, axis=1, keepdims=True)
    var = jnp.mean((h - mu) ** 2, axis=1, keepdims=True)
    return (h - mu) / jnp.sqrt(var + 1e-5) * s_ref[...] + b_ref[...]


def _mid_body(a0, a1, d0, d1, x, rw, rb, lns, lnb, wcat, h_out, z_out):
    deg = d0[:, 0:1] + d1[:, 0:1]
    dinv = jnp.where(deg > 0, 1.0 / deg, 0.0)
    h = dinv * (a0[...] + a1[...])
    h = h + jnp.dot(x[...], rw[...], preferred_element_type=jnp.float32) + rb[...]
    h = jnp.maximum(_layer_tail(h, lns, lnb), 0.0)
    h_out[...] = h
    z_out[...] = jnp.dot(h, wcat[...], preferred_element_type=jnp.float32)


def _mid(a0, a1, d0, d1, x, rootW, rootb, lns, lnb, Wcat1):
    return pl.pallas_call(
        _mid_body,
        grid=(N // RB,),
        in_specs=[
            pl.BlockSpec((RB, H), lambda i: (i, 0)),
            pl.BlockSpec((RB, H), lambda i: (i, 0)),
            pl.BlockSpec((RB, 16), lambda i: (i, 0)),
            pl.BlockSpec((RB, 16), lambda i: (i, 0)),
            pl.BlockSpec((RB, H), lambda i: (i, 0)),
            pl.BlockSpec((H, H), lambda i: (0, 0)),
            pl.BlockSpec((1, H), lambda i: (0, 0)),
            pl.BlockSpec((1, H), lambda i: (0, 0)),
            pl.BlockSpec((1, H), lambda i: (0, 0)),
            pl.BlockSpec((H, R * H), lambda i: (0, 0)),
        ],
        out_specs=[
            pl.BlockSpec((RB, H), lambda i: (i, 0)),
            pl.BlockSpec((RB, R * H), lambda i: (i, 0)),
        ],
        out_shape=[
            jax.ShapeDtypeStruct((N, H), jnp.float32),
            jax.ShapeDtypeStruct((N, R * H), jnp.float32),
        ],
    )(a0, a1, d0, d1, x, rootW, rootb, lns, lnb, Wcat1)


def _fin_body(a0, a1, d0, d1, h, rw, rb, lns, lnb, out):
    deg = d0[:, 0:1] + d1[:, 0:1]
    dinv = jnp.where(deg > 0, 1.0 / deg, 0.0)
    o = dinv * (a0[...] + a1[...])
    o = o + jnp.dot(h[...], rw[...], preferred_element_type=jnp.float32) + rb[...]
    out[...] = _layer_tail(o, lns, lnb)


def _final(a0, a1, d0, d1, h, rootW, rootb, lns, lnb):
    return pl.pallas_call(
        _fin_body,
        grid=(N // RB,),
        in_specs=[
            pl.BlockSpec((RB, H), lambda i: (i, 0)),
            pl.BlockSpec((RB, H), lambda i: (i, 0)),
            pl.BlockSpec((RB, 16), lambda i: (i, 0)),
            pl.BlockSpec((RB, 16), lambda i: (i, 0)),
            pl.BlockSpec((RB, H), lambda i: (i, 0)),
            pl.BlockSpec((H, H), lambda i: (0, 0)),
            pl.BlockSpec((1, H), lambda i: (0, 0)),
            pl.BlockSpec((1, H), lambda i: (0, 0)),
            pl.BlockSpec((1, H), lambda i: (0, 0)),
        ],
        out_specs=pl.BlockSpec((RB, H), lambda i: (i, 0)),
        out_shape=jax.ShapeDtypeStruct((N, H), jnp.float32),
    )(a0, a1, d0, d1, h, rootW, rootb, lns, lnb)


def _make_sc_agg(with_deg):
    mesh = plsc.VectorSubcoreMesh(
        core_axis_name="c", subcore_axis_name="s", num_cores=NC)
    out_type = [jax.ShapeDtypeStruct((NC, N_PAD, H), jnp.float32)]
    scratch = [
        pltpu.VMEM((CH, S), jnp.int32),
        pltpu.VMEM((CH, S), jnp.int32),
        pltpu.VMEM((CH, S), jnp.int32),
        pltpu.VMEM((CH, S), jnp.int32),
        pltpu.VMEM((S, H), jnp.float32),
        pltpu.VMEM((ZR, H), jnp.float32),
        pltpu.VMEM_SHARED((N_PAD, H), jnp.float32),
        pltpu.SemaphoreType.DMA,
    ]
    if with_deg:
        out_type.append(jax.ShapeDtypeStruct((NC, N_PAD, 16), jnp.float32))
        scratch += [
            pltpu.VMEM((S, 16), jnp.float32),
            pltpu.VMEM((ZR, 16), jnp.float32),
            pltpu.VMEM_SHARED((N_PAD, 16), jnp.float32),
        ]

    def body(table, srcs, rels, dsts, *rest):
        if with_deg:
            (agg_out, deg_out, src_v, rel_v, dst_v, idx_v, rows_v, zrow_v,
             acc_sh, sem, ones_v, zdeg_v, deg_sh) = rest
        else:
            (agg_out, src_v, rel_v, dst_v, idx_v, rows_v, zrow_v,
             acc_sh, sem) = rest
        cid = lax.axis_index("c")
        sid = lax.axis_index("s")
        wid = sid * NC + cid
        base = sid * RPT

        pltpu.sync_copy(srcs.at[wid], src_v)
        pltpu.sync_copy(rels.at[wid], rel_v)
        pltpu.sync_copy(dsts.at[wid], dst_v)

        z16 = jnp.zeros((16,), jnp.float32)

        def zfill(i, _):
            for j in range(H // 16):
                zrow_v[i, pl.ds(j * 16, 16)] = z16
            if with_deg:
                zdeg_v[i, :] = z16
            return 0

        lax.fori_loop(0, ZR, zfill, 0)
        for k in range(RPT // ZR):
            pltpu.sync_copy(zrow_v, acc_sh.at[pl.ds(base + k * ZR, ZR)])
            if with_deg:
                pltpu.sync_copy(zdeg_v, deg_sh.at[pl.ds(base + k * ZR, ZR)])

        if with_deg:
            o16 = jnp.ones((16,), jnp.float32)

            def ofill(i, _):
                ones_v[i, :] = o16
                return 0

            lax.fori_loop(0, S, ofill, 0)

        def idx_fill(c, _):
            for j in range(S // 16):
                sl = pl.ds(j * 16, 16)
                idx_v[c, sl] = src_v[c, sl] * R + rel_v[c, sl]
            return 0

        lax.fori_loop(0, CH, idx_fill, 0)

        plsc.subcore_barrier()

        def chunk(c, _):
            pltpu.async_copy(table.at[idx_v.at[c]], rows_v, sem).wait()
            pltpu.sync_copy(rows_v, acc_sh.at[dst_v.at[c]], add=True)
            if with_deg:
                pltpu.sync_copy(ones_v, deg_sh.at[dst_v.at[c]], add=True)
            return 0

        lax.fori_loop(0, CH, chunk, 0)

        plsc.subcore_barrier()

        pltpu.sync_copy(acc_sh.at[pl.ds(base, RPT)],
                        agg_out.at[cid, pl.ds(base, RPT)])
        if with_deg:
            pltpu.sync_copy(deg_sh.at[pl.ds(base, RPT)],
                            deg_out.at[cid, pl.ds(base, RPT)])

    return functools.partial(
        pl.kernel, mesh=mesh,
        out_type=tuple(out_type) if with_deg else out_type[0],
        scratch_types=scratch,
        compiler_params=pltpu.CompilerParams(use_tc_tiling_on_sc=False),
    )(body)


_sc_cache = {}


def _sc_agg_kernel(with_deg):
    if with_deg not in _sc_cache:
        _sc_cache[with_deg] = _make_sc_agg(with_deg)
    return _sc_cache[with_deg]


def kernel(node_features, edge_triples, num_nodes, W_in, b_in, basis0, att0,
           rootW0, rootb0, ln_s0, ln_b0, basis1, att1, rootW1, rootb1,
           ln_s1, ln_b1):
    src = edge_triples[:, 0].astype(jnp.int32).reshape(NW, CH, S)
    rel = edge_triples[:, 1].astype(jnp.int32).reshape(NW, CH, S)
    dst = edge_triples[:, 2].astype(jnp.int32).reshape(NW, CH, S)
    Wcat0 = jnp.einsum('rb,bio->iro', att0, basis0).reshape(H, R * H)
    Wcat1 = jnp.einsum('rb,bio->iro', att1, basis1).reshape(H, R * H)

    x, z0 = _encode(node_features, W_in, b_in.reshape(1, H), Wcat0)
    agg0, deg = _sc_agg_kernel(True)(z0.reshape(N * R, H), src, rel, dst)
    h, z1 = _mid(agg0[0], agg0[1], deg[0], deg[1], x, rootW0,
                 rootb0.reshape(1, H), ln_s0.reshape(1, H),
                 ln_b0.reshape(1, H), Wcat1)
    agg1 = _sc_agg_kernel(False)(z1.reshape(N * R, H), src, rel, dst)
    out = _final(agg1[0], agg1[1], deg[0], deg[1], h, rootW1,
                 rootb1.reshape(1, H), ln_s1.reshape(1, H),
                 ln_b1.reshape(1, H))
    return out


# speedup vs baseline: 25.5625x; 1.4545x over previous
import functools

import jax
import jax.numpy as jnp
from jax import lax
from jax.experimental import pallas as pl
from jax.experimental.pallas import tpu as pltpu
from jax.experimental.pallas import tpu_sc as plsc

N = 10000
E = 320000
R = 8
H = 64

NC = 2
NS = 16
NW = NC * NS
EPW = E // NW
S = 80
CH = EPW // S
GRP = 5
N_PAD = 10240
RPT = N_PAD // NS
ZR = 128

RB = 2000


def _enc_body(nf, win, bin_, wcat, x_out, z_out):
    x = jnp.dot(nf[...], win[...], preferred_element_type=jnp.float32) + bin_[...]
    x_out[...] = x
    z_out[...] = jnp.dot(x, wcat[...], preferred_element_type=jnp.float32)


def _encode(nf, W_in, b_in, Wcat0):
    return pl.pallas_call(
        _enc_body,
        grid=(N // RB,),
        in_specs=[
            pl.BlockSpec((RB, 128), lambda i: (i, 0)),
            pl.BlockSpec((128, H), lambda i: (0, 0)),
            pl.BlockSpec((1, H), lambda i: (0, 0)),
            pl.BlockSpec((H, R * H), lambda i: (0, 0)),
        ],
        out_specs=[
            pl.BlockSpec((RB, H), lambda i: (i, 0)),
            pl.BlockSpec((RB, R * H), lambda i: (i, 0)),
        ],
        out_shape=[
            jax.ShapeDtypeStruct((N, H), jnp.float32),
            jax.ShapeDtypeStruct((N, R * H), jnp.float32),
        ],
    )(nf, W_in, b_in, Wcat0)


def _layer_tail(h, s_ref, b_ref):
    mu = jnp.mean(h, axis=1, keepdims=True)
    var = jnp.mean((h - mu) ** 2, axis=1, keepdims=True)
    return (h - mu) / jnp.sqrt(var + 1e-5) * s_ref[...] + b_ref[...]


def _mid_body(a0, a1, d0, d1, x, rw, rb, lns, lnb, wcat, h_out, z_out):
    deg = d0[:, 0:1] + d1[:, 0:1]
    dinv = jnp.where(deg > 0, 1.0 / deg, 0.0)
    h = dinv * (a0[...] + a1[...])
    h = h + jnp.dot(x[...], rw[...], preferred_element_type=jnp.float32) + rb[...]
    h = jnp.maximum(_layer_tail(h, lns, lnb), 0.0)
    h_out[...] = h
    z_out[...] = jnp.dot(h, wcat[...], preferred_element_type=jnp.float32)


def _mid(a0, a1, d0, d1, x, rootW, rootb, lns, lnb, Wcat1):
    return pl.pallas_call(
        _mid_body,
        grid=(N // RB,),
        in_specs=[
            pl.BlockSpec((RB, H), lambda i: (i, 0)),
            pl.BlockSpec((RB, H), lambda i: (i, 0)),
            pl.BlockSpec((RB, 16), lambda i: (i, 0)),
            pl.BlockSpec((RB, 16), lambda i: (i, 0)),
            pl.BlockSpec((RB, H), lambda i: (i, 0)),
            pl.BlockSpec((H, H), lambda i: (0, 0)),
            pl.BlockSpec((1, H), lambda i: (0, 0)),
            pl.BlockSpec((1, H), lambda i: (0, 0)),
            pl.BlockSpec((1, H), lambda i: (0, 0)),
            pl.BlockSpec((H, R * H), lambda i: (0, 0)),
        ],
        out_specs=[
            pl.BlockSpec((RB, H), lambda i: (i, 0)),
            pl.BlockSpec((RB, R * H), lambda i: (i, 0)),
        ],
        out_shape=[
            jax.ShapeDtypeStruct((N, H), jnp.float32),
            jax.ShapeDtypeStruct((N, R * H), jnp.float32),
        ],
    )(a0, a1, d0, d1, x, rootW, rootb, lns, lnb, Wcat1)


def _fin_body(a0, a1, d0, d1, h, rw, rb, lns, lnb, out):
    deg = d0[:, 0:1] + d1[:, 0:1]
    dinv = jnp.where(deg > 0, 1.0 / deg, 0.0)
    o = dinv * (a0[...] + a1[...])
    o = o + jnp.dot(h[...], rw[...], preferred_element_type=jnp.float32) + rb[...]
    out[...] = _layer_tail(o, lns, lnb)


def _final(a0, a1, d0, d1, h, rootW, rootb, lns, lnb):
    return pl.pallas_call(
        _fin_body,
        grid=(N // RB,),
        in_specs=[
            pl.BlockSpec((RB, H), lambda i: (i, 0)),
            pl.BlockSpec((RB, H), lambda i: (i, 0)),
            pl.BlockSpec((RB, 16), lambda i: (i, 0)),
            pl.BlockSpec((RB, 16), lambda i: (i, 0)),
            pl.BlockSpec((RB, H), lambda i: (i, 0)),
            pl.BlockSpec((H, H), lambda i: (0, 0)),
            pl.BlockSpec((1, H), lambda i: (0, 0)),
            pl.BlockSpec((1, H), lambda i: (0, 0)),
            pl.BlockSpec((1, H), lambda i: (0, 0)),
        ],
        out_specs=pl.BlockSpec((RB, H), lambda i: (i, 0)),
        out_shape=jax.ShapeDtypeStruct((N, H), jnp.float32),
    )(a0, a1, d0, d1, h, rootW, rootb, lns, lnb)


def _make_sc_agg(with_deg):
    mesh = plsc.VectorSubcoreMesh(
        core_axis_name="c", subcore_axis_name="s", num_cores=NC)
    out_type = [jax.ShapeDtypeStruct((NC, N_PAD, H), jnp.float32)]
    scratch = [
        pltpu.VMEM((CH, S), jnp.int32),
        pltpu.VMEM((CH, S), jnp.int32),
        pltpu.VMEM((CH, S), jnp.int32),
        pltpu.VMEM((GRP * S, H), jnp.float32),
        pltpu.VMEM((ZR, H), jnp.float32),
        pltpu.VMEM_SHARED((N_PAD, H), jnp.float32),
        [pltpu.SemaphoreType.DMA] * GRP,
        pltpu.SemaphoreType.DMA,
    ]
    if with_deg:
        out_type.append(jax.ShapeDtypeStruct((NC, N_PAD, 16), jnp.float32))
        scratch += [
            pltpu.VMEM((S, 16), jnp.float32),
            pltpu.VMEM((ZR, 16), jnp.float32),
            pltpu.VMEM_SHARED((N_PAD, 16), jnp.float32),
            pltpu.SemaphoreType.DMA,
        ]

    def body(table, srcs, rels, dsts, *rest):
        if with_deg:
            (agg_out, deg_out, idx_v, rel_v, dst_v, rows_v, zrow_v,
             acc_sh, gsems, ssem, ones_v, zdeg_v, deg_sh, dsem) = rest
        else:
            (agg_out, idx_v, rel_v, dst_v, rows_v, zrow_v,
             acc_sh, gsems, ssem) = rest
        cid = lax.axis_index("c")
        sid = lax.axis_index("s")
        wid = sid * NC + cid
        base = sid * RPT

        pltpu.sync_copy(srcs.at[wid], idx_v)
        pltpu.sync_copy(rels.at[wid], rel_v)
        pltpu.sync_copy(dsts.at[wid], dst_v)

        z16 = jnp.zeros((16,), jnp.float32)

        def zfill(i, _):
            for j in range(H // 16):
                zrow_v[i, pl.ds(j * 16, 16)] = z16
            if with_deg:
                zdeg_v[i, :] = z16
            return 0

        lax.fori_loop(0, ZR, zfill, 0)
        for k in range(RPT // ZR):
            pltpu.sync_copy(zrow_v, acc_sh.at[pl.ds(base + k * ZR, ZR)])
            if with_deg:
                pltpu.sync_copy(zdeg_v, deg_sh.at[pl.ds(base + k * ZR, ZR)])

        if with_deg:
            o16 = jnp.ones((16,), jnp.float32)

            def ofill(i, _):
                ones_v[i, :] = o16
                return 0

            lax.fori_loop(0, S, ofill, 0)

        def idx_fill(c, _):
            for j in range(S // 16):
                sl = pl.ds(j * 16, 16)
                idx_v[c, sl] = idx_v[c, sl] * R + rel_v[c, sl]
            return 0

        lax.fori_loop(0, CH, idx_fill, 0)

        plsc.subcore_barrier()

        def group(g, _):
            c0 = g * GRP
            gets = [
                pltpu.async_copy(table.at[idx_v.at[c0 + b]],
                                 rows_v.at[pl.ds(b * S, S)], gsems[b])
                for b in range(GRP)
            ]
            puts = []
            for b in range(GRP):
                gets[b].wait()
                puts.append(pltpu.async_copy(
                    rows_v.at[pl.ds(b * S, S)],
                    acc_sh.at[dst_v.at[c0 + b]], ssem, add=True))
                if with_deg:
                    puts.append(pltpu.async_copy(
                        ones_v, deg_sh.at[dst_v.at[c0 + b]], dsem, add=True))
            for p in puts:
                p.wait()
            return 0

        lax.fori_loop(0, CH // GRP, group, 0)

        plsc.subcore_barrier()

        pltpu.sync_copy(acc_sh.at[pl.ds(base, RPT)],
                        agg_out.at[cid, pl.ds(base, RPT)])
        if with_deg:
            pltpu.sync_copy(deg_sh.at[pl.ds(base, RPT)],
                            deg_out.at[cid, pl.ds(base, RPT)])

    return functools.partial(
        pl.kernel, mesh=mesh,
        out_type=tuple(out_type) if with_deg else out_type[0],
        scratch_types=scratch,
        compiler_params=pltpu.CompilerParams(use_tc_tiling_on_sc=False),
    )(body)


_sc_cache = {}


def _sc_agg_kernel(with_deg):
    if with_deg not in _sc_cache:
        _sc_cache[with_deg] = _make_sc_agg(with_deg)
    return _sc_cache[with_deg]


def kernel(node_features, edge_triples, num_nodes, W_in, b_in, basis0, att0,
           rootW0, rootb0, ln_s0, ln_b0, basis1, att1, rootW1, rootb1,
           ln_s1, ln_b1):
    src = edge_triples[:, 0].astype(jnp.int32).reshape(NW, CH, S)
    rel = edge_triples[:, 1].astype(jnp.int32).reshape(NW, CH, S)
    dst = edge_triples[:, 2].astype(jnp.int32).reshape(NW, CH, S)
    Wcat0 = jnp.einsum('rb,bio->iro', att0, basis0).reshape(H, R * H)
    Wcat1 = jnp.einsum('rb,bio->iro', att1, basis1).reshape(H, R * H)

    x, z0 = _encode(node_features, W_in, b_in.reshape(1, H), Wcat0)
    agg0, deg = _sc_agg_kernel(True)(z0.reshape(N * R, H), src, rel, dst)
    h, z1 = _mid(agg0[0], agg0[1], deg[0], deg[1], x, rootW0,
                 rootb0.reshape(1, H), ln_s0.reshape(1, H),
                 ln_b0.reshape(1, H), Wcat1)
    agg1 = _sc_agg_kernel(False)(z1.reshape(N * R, H), src, rel, dst)
    out = _final(agg1[0], agg1[1], deg[0], deg[1], h, rootW1,
                 rootb1.reshape(1, H), ln_s1.reshape(1, H),
                 ln_b1.reshape(1, H))
    return out


# speedup vs baseline: 27.8754x; 1.0905x over previous
import functools

import jax
import jax.numpy as jnp
from jax import lax
from jax.experimental import pallas as pl
from jax.experimental.pallas import tpu as pltpu
from jax.experimental.pallas import tpu_sc as plsc

N = 10000
E = 320000
R = 8
H = 64

NC = 2
NS = 16
NW = NC * NS
EPW = E // NW
S = 80
CH = EPW // S
GRP = 5
SEG = 25
N_PAD = 10240
RPT = N_PAD // NS
ZR = 128

RB = 2000


def _enc_body(nf, win, bin_, wcat, x_out, z_out):
    x = jnp.dot(nf[...], win[...], preferred_element_type=jnp.float32) + bin_[...]
    x_out[...] = x
    z_out[...] = jnp.dot(x, wcat[...], preferred_element_type=jnp.float32)


def _encode(nf, W_in, b_in, Wcat0):
    return pl.pallas_call(
        _enc_body,
        grid=(N // RB,),
        in_specs=[
            pl.BlockSpec((RB, 128), lambda i: (i, 0)),
            pl.BlockSpec((128, H), lambda i: (0, 0)),
            pl.BlockSpec((1, H), lambda i: (0, 0)),
            pl.BlockSpec((H, R * H), lambda i: (0, 0)),
        ],
        out_specs=[
            pl.BlockSpec((RB, H), lambda i: (i, 0)),
            pl.BlockSpec((RB, R * H), lambda i: (i, 0)),
        ],
        out_shape=[
            jax.ShapeDtypeStruct((N, H), jnp.float32),
            jax.ShapeDtypeStruct((N, R * H), jnp.float32),
        ],
    )(nf, W_in, b_in, Wcat0)


def _layer_tail(h, s_ref, b_ref):
    mu = jnp.mean(h, axis=1, keepdims=True)
    var = jnp.mean((h - mu) ** 2, axis=1, keepdims=True)
    return (h - mu) / jnp.sqrt(var + 1e-5) * s_ref[...] + b_ref[...]


def _mid_body(a0, a1, d0, d1, x, rw, rb, lns, lnb, wcat, h_out, z_out):
    deg = d0[:, 0:1] + d1[:, 0:1]
    dinv = jnp.where(deg > 0, 1.0 / deg, 0.0)
    h = dinv * (a0[...] + a1[...])
    h = h + jnp.dot(x[...], rw[...], preferred_element_type=jnp.float32) + rb[...]
    h = jnp.maximum(_layer_tail(h, lns, lnb), 0.0)
    h_out[...] = h
    z_out[...] = jnp.dot(h, wcat[...], preferred_element_type=jnp.float32)


def _mid(a0, a1, d0, d1, x, rootW, rootb, lns, lnb, Wcat1):
    return pl.pallas_call(
        _mid_body,
        grid=(N // RB,),
        in_specs=[
            pl.BlockSpec((RB, H), lambda i: (i, 0)),
            pl.BlockSpec((RB, H), lambda i: (i, 0)),
            pl.BlockSpec((RB, 16), lambda i: (i, 0)),
            pl.BlockSpec((RB, 16), lambda i: (i, 0)),
            pl.BlockSpec((RB, H), lambda i: (i, 0)),
            pl.BlockSpec((H, H), lambda i: (0, 0)),
            pl.BlockSpec((1, H), lambda i: (0, 0)),
            pl.BlockSpec((1, H), lambda i: (0, 0)),
            pl.BlockSpec((1, H), lambda i: (0, 0)),
            pl.BlockSpec((H, R * H), lambda i: (0, 0)),
        ],
        out_specs=[
            pl.BlockSpec((RB, H), lambda i: (i, 0)),
            pl.BlockSpec((RB, R * H), lambda i: (i, 0)),
        ],
        out_shape=[
            jax.ShapeDtypeStruct((N, H), jnp.float32),
            jax.ShapeDtypeStruct((N, R * H), jnp.float32),
        ],
    )(a0, a1, d0, d1, x, rootW, rootb, lns, lnb, Wcat1)


def _fin_body(a0, a1, d0, d1, h, rw, rb, lns, lnb, out):
    deg = d0[:, 0:1] + d1[:, 0:1]
    dinv = jnp.where(deg > 0, 1.0 / deg, 0.0)
    o = dinv * (a0[...] + a1[...])
    o = o + jnp.dot(h[...], rw[...], preferred_element_type=jnp.float32) + rb[...]
    out[...] = _layer_tail(o, lns, lnb)


def _final(a0, a1, d0, d1, h, rootW, rootb, lns, lnb):
    return pl.pallas_call(
        _fin_body,
        grid=(N // RB,),
        in_specs=[
            pl.BlockSpec((RB, H), lambda i: (i, 0)),
            pl.BlockSpec((RB, H), lambda i: (i, 0)),
            pl.BlockSpec((RB, 16), lambda i: (i, 0)),
            pl.BlockSpec((RB, 16), lambda i: (i, 0)),
            pl.BlockSpec((RB, H), lambda i: (i, 0)),
            pl.BlockSpec((H, H), lambda i: (0, 0)),
            pl.BlockSpec((1, H), lambda i: (0, 0)),
            pl.BlockSpec((1, H), lambda i: (0, 0)),
            pl.BlockSpec((1, H), lambda i: (0, 0)),
        ],
        out_specs=pl.BlockSpec((RB, H), lambda i: (i, 0)),
        out_shape=jax.ShapeDtypeStruct((N, H), jnp.float32),
    )(a0, a1, d0, d1, h, rootW, rootb, lns, lnb)


def _make_sc_agg(with_deg):
    mesh = plsc.VectorSubcoreMesh(
        core_axis_name="c", subcore_axis_name="s", num_cores=NC)
    out_type = [jax.ShapeDtypeStruct((NC, N_PAD, H), jnp.float32)]
    scratch = [
        pltpu.VMEM((CH, S), jnp.int32),
        pltpu.VMEM((SEG, S), jnp.int32),
        pltpu.VMEM((CH, S), jnp.int32),
        pltpu.VMEM((2 * GRP * S, H), jnp.float32),
        pltpu.VMEM_SHARED((N_PAD, H), jnp.float32),
        [pltpu.SemaphoreType.DMA] * (2 * GRP),
        [pltpu.SemaphoreType.DMA] * 2,
    ]
    if with_deg:
        out_type.append(jax.ShapeDtypeStruct((NC, N_PAD, 16), jnp.float32))
        scratch += [
            pltpu.VMEM((S, 16), jnp.float32),
            pltpu.VMEM((ZR, 16), jnp.float32),
            pltpu.VMEM_SHARED((N_PAD, 16), jnp.float32),
            pltpu.SemaphoreType.DMA,
        ]

    def body(table, srcs, rels, dsts, *rest):
        if with_deg:
            (agg_out, deg_out, idx_v, rseg_v, dst_v, rows_v,
             acc_sh, gsems, ssems, ones_v, zdeg_v, deg_sh, dsem) = rest
        else:
            (agg_out, idx_v, rseg_v, dst_v, rows_v,
             acc_sh, gsems, ssems) = rest
        cid = lax.axis_index("c")
        sid = lax.axis_index("s")
        wid = sid * NC + cid
        base = sid * RPT

        pltpu.sync_copy(srcs.at[wid], idx_v)
        pltpu.sync_copy(dsts.at[wid], dst_v)

        for sg in range(CH // SEG):
            pltpu.sync_copy(rels.at[wid, pl.ds(sg * SEG, SEG)], rseg_v)

            def idx_fill(c, _):
                for j in range(S // 16):
                    sl = pl.ds(j * 16, 16)
                    idx_v[sg * SEG + c, sl] = (
                        idx_v[sg * SEG + c, sl] * R + rseg_v[c, sl])
                return 0

            lax.fori_loop(0, SEG, idx_fill, 0)

        z16 = jnp.zeros((16,), jnp.float32)

        def zfill(i, _):
            for j in range(H // 16):
                rows_v[i, pl.ds(j * 16, 16)] = z16
            if with_deg:
                zdeg_v[i, :] = z16
            return 0

        lax.fori_loop(0, ZR, zfill, 0)
        for k in range(RPT // ZR):
            pltpu.sync_copy(rows_v.at[pl.ds(0, ZR)],
                            acc_sh.at[pl.ds(base + k * ZR, ZR)])
            if with_deg:
                pltpu.sync_copy(zdeg_v, deg_sh.at[pl.ds(base + k * ZR, ZR)])

        if with_deg:
            o16 = jnp.ones((16,), jnp.float32)

            def ofill(i, _):
                ones_v[i, :] = o16
                return 0

            lax.fori_loop(0, S, ofill, 0)

        plsc.subcore_barrier()

        def drain_rows(half):
            for b in range(GRP):
                pltpu.make_async_copy(
                    table.at[pl.ds(0, S)],
                    rows_v.at[pl.ds((half * GRP + b) * S, S)],
                    ssems[half]).wait()

        def drain_deg(count):
            for _ in range(count):
                pltpu.make_async_copy(
                    table.at[pl.ds(0, S), pl.ds(0, 16)], ones_v, dsem).wait()

        def fire_group(g, half):
            c0 = g * GRP
            off = half * GRP * S
            gets = [
                pltpu.async_copy(table.at[idx_v.at[c0 + b]],
                                 rows_v.at[pl.ds(off + b * S, S)],
                                 gsems[half * GRP + b])
                for b in range(GRP)
            ]
            for b in range(GRP):
                gets[b].wait()
                pltpu.async_copy(rows_v.at[pl.ds(off + b * S, S)],
                                 acc_sh.at[dst_v.at[c0 + b]],
                                 ssems[half], add=True)
                if with_deg:
                    pltpu.async_copy(ones_v, deg_sh.at[dst_v.at[c0 + b]],
                                     dsem, add=True)

        def pair(p, _):
            @pl.when(p > 0)
            def _():
                drain_rows(0)
                if with_deg:
                    drain_deg(2 * GRP)
            fire_group(2 * p, 0)

            @pl.when(p > 0)
            def _():
                drain_rows(1)
            fire_group(2 * p + 1, 1)
            return 0

        npair = (CH // GRP) // 2
        lax.fori_loop(0, npair, pair, 0)
        drain_rows(0)
        fire_group(CH // GRP - 1, 0)
        drain_rows(0)
        drain_rows(1)
        if with_deg:
            drain_deg((CH // GRP - (npair - 1) * 2) * GRP)

        plsc.subcore_barrier()

        pltpu.sync_copy(acc_sh.at[pl.ds(base, RPT)],
                        agg_out.at[cid, pl.ds(base, RPT)])
        if with_deg:
            pltpu.sync_copy(deg_sh.at[pl.ds(base, RPT)],
                            deg_out.at[cid, pl.ds(base, RPT)])

    return functools.partial(
        pl.kernel, mesh=mesh,
        out_type=tuple(out_type) if with_deg else out_type[0],
        scratch_types=scratch,
        compiler_params=pltpu.CompilerParams(use_tc_tiling_on_sc=False),
    )(body)


_sc_cache = {}


def _sc_agg_kernel(with_deg):
    if with_deg not in _sc_cache:
        _sc_cache[with_deg] = _make_sc_agg(with_deg)
    return _sc_cache[with_deg]


def kernel(node_features, edge_triples, num_nodes, W_in, b_in, basis0, att0,
           rootW0, rootb0, ln_s0, ln_b0, basis1, att1, rootW1, rootb1,
           ln_s1, ln_b1):
    src = edge_triples[:, 0].astype(jnp.int32).reshape(NW, CH, S)
    rel = edge_triples[:, 1].astype(jnp.int32).reshape(NW, CH, S)
    dst = edge_triples[:, 2].astype(jnp.int32).reshape(NW, CH, S)
    Wcat0 = jnp.einsum('rb,bio->iro', att0, basis0).reshape(H, R * H)
    Wcat1 = jnp.einsum('rb,bio->iro', att1, basis1).reshape(H, R * H)

    x, z0 = _encode(node_features, W_in, b_in.reshape(1, H), Wcat0)
    agg0, deg = _sc_agg_kernel(True)(z0.reshape(N * R, H), src, rel, dst)
    h, z1 = _mid(agg0[0], agg0[1], deg[0], deg[1], x, rootW0,
                 rootb0.reshape(1, H), ln_s0.reshape(1, H),
                 ln_b0.reshape(1, H), Wcat1)
    agg1 = _sc_agg_kernel(False)(z1.reshape(N * R, H), src, rel, dst)
    out = _final(agg1[0], agg1[1], deg[0], deg[1], h, rootW1,
                 rootb1.reshape(1, H), ln_s1.reshape(1, H),
                 ln_b1.reshape(1, H))
    return out


# speedup vs baseline: 30.2163x; 1.0840x over previous
import functools

import jax
import jax.numpy as jnp
from jax import lax
from jax.experimental import pallas as pl
from jax.experimental.pallas import tpu as pltpu
from jax.experimental.pallas import tpu_sc as plsc

N = 10000
E = 320000
R = 8
H = 64

NC = 2
NS = 16
NW = NC * NS
EPW = E // NW
S = 80
CH = EPW // S
GRP = 5
SEG = 25
N_PAD = 10240
RPT = N_PAD // NS
ZR = 128

RB = 2000


def _enc_body(nf, win, bin_, wcat, x_out, z_out):
    x = jnp.dot(nf[...], win[...], preferred_element_type=jnp.float32) + bin_[...]
    x_out[...] = x
    for q in range(R * H // 128):
        z_out[q] = jnp.dot(x, wcat[:, 128 * q:128 * (q + 1)],
                           preferred_element_type=jnp.float32)


def _encode(nf, W_in, b_in, Wcat0):
    return pl.pallas_call(
        _enc_body,
        grid=(N // RB,),
        in_specs=[
            pl.BlockSpec((RB, 128), lambda i: (i, 0)),
            pl.BlockSpec((128, H), lambda i: (0, 0)),
            pl.BlockSpec((1, H), lambda i: (0, 0)),
            pl.BlockSpec((H, R * H), lambda i: (0, 0)),
        ],
        out_specs=[
            pl.BlockSpec((RB, H), lambda i: (i, 0)),
            pl.BlockSpec((R * H // 128, RB, 128), lambda i: (0, i, 0)),
        ],
        out_shape=[
            jax.ShapeDtypeStruct((N, H), jnp.float32),
            jax.ShapeDtypeStruct((R * H // 128, N, 128), jnp.float32),
        ],
    )(nf, W_in, b_in, Wcat0)


def _layer_tail(h, s_ref, b_ref):
    mu = jnp.mean(h, axis=1, keepdims=True)
    var = jnp.mean((h - mu) ** 2, axis=1, keepdims=True)
    return (h - mu) / jnp.sqrt(var + 1e-5) * s_ref[...] + b_ref[...]


def _mid_body(a0, a1, d0, d1, x, rw, rb, lns, lnb, wcat, h_out, z_out):
    deg = d0[:, 0:1] + d1[:, 0:1]
    dinv = jnp.where(deg > 0, 1.0 / deg, 0.0)
    h = dinv * (a0[...] + a1[...])
    h = h + jnp.dot(x[...], rw[...], preferred_element_type=jnp.float32) + rb[...]
    h = jnp.maximum(_layer_tail(h, lns, lnb), 0.0)
    h_out[...] = h
    for q in range(R * H // 128):
        z_out[q] = jnp.dot(h, wcat[:, 128 * q:128 * (q + 1)],
                           preferred_element_type=jnp.float32)


def _mid(a0, a1, d0, d1, x, rootW, rootb, lns, lnb, Wcat1):
    return pl.pallas_call(
        _mid_body,
        grid=(N // RB,),
        in_specs=[
            pl.BlockSpec((RB, H), lambda i: (i, 0)),
            pl.BlockSpec((RB, H), lambda i: (i, 0)),
            pl.BlockSpec((RB, 16), lambda i: (i, 0)),
            pl.BlockSpec((RB, 16), lambda i: (i, 0)),
            pl.BlockSpec((RB, H), lambda i: (i, 0)),
            pl.BlockSpec((H, H), lambda i: (0, 0)),
            pl.BlockSpec((1, H), lambda i: (0, 0)),
            pl.BlockSpec((1, H), lambda i: (0, 0)),
            pl.BlockSpec((1, H), lambda i: (0, 0)),
            pl.BlockSpec((H, R * H), lambda i: (0, 0)),
        ],
        out_specs=[
            pl.BlockSpec((RB, H), lambda i: (i, 0)),
            pl.BlockSpec((R * H // 128, RB, 128), lambda i: (0, i, 0)),
        ],
        out_shape=[
            jax.ShapeDtypeStruct((N, H), jnp.float32),
            jax.ShapeDtypeStruct((R * H // 128, N, 128), jnp.float32),
        ],
    )(a0, a1, d0, d1, x, rootW, rootb, lns, lnb, Wcat1)


def _fin_body(a0, a1, d0, d1, h, rw, rb, lns, lnb, out):
    deg = d0[:, 0:1] + d1[:, 0:1]
    dinv = jnp.where(deg > 0, 1.0 / deg, 0.0)
    o = dinv * (a0[...] + a1[...])
    o = o + jnp.dot(h[...], rw[...], preferred_element_type=jnp.float32) + rb[...]
    out[...] = _layer_tail(o, lns, lnb)


def _final(a0, a1, d0, d1, h, rootW, rootb, lns, lnb):
    return pl.pallas_call(
        _fin_body,
        grid=(N // RB,),
        in_specs=[
            pl.BlockSpec((RB, H), lambda i: (i, 0)),
            pl.BlockSpec((RB, H), lambda i: (i, 0)),
            pl.BlockSpec((RB, 16), lambda i: (i, 0)),
            pl.BlockSpec((RB, 16), lambda i: (i, 0)),
            pl.BlockSpec((RB, H), lambda i: (i, 0)),
            pl.BlockSpec((H, H), lambda i: (0, 0)),
            pl.BlockSpec((1, H), lambda i: (0, 0)),
            pl.BlockSpec((1, H), lambda i: (0, 0)),
            pl.BlockSpec((1, H), lambda i: (0, 0)),
        ],
        out_specs=pl.BlockSpec((RB, H), lambda i: (i, 0)),
        out_shape=jax.ShapeDtypeStruct((N, H), jnp.float32),
    )(a0, a1, d0, d1, h, rootW, rootb, lns, lnb)


def _make_sc_agg(with_deg):
    mesh = plsc.VectorSubcoreMesh(
        core_axis_name="c", subcore_axis_name="s", num_cores=NC)
    out_type = [jax.ShapeDtypeStruct((NC, N_PAD, H), jnp.float32)]
    scratch = [
        pltpu.VMEM((CH, S), jnp.int32),
        pltpu.VMEM((SEG, S), jnp.int32),
        pltpu.VMEM((CH, S), jnp.int32),
        pltpu.VMEM((2 * GRP * S, H), jnp.float32),
        pltpu.VMEM_SHARED((N_PAD, H), jnp.float32),
        [pltpu.SemaphoreType.DMA] * (2 * GRP),
        [pltpu.SemaphoreType.DMA] * 2,
    ]
    if with_deg:
        out_type.append(jax.ShapeDtypeStruct((NC, N_PAD, 16), jnp.float32))
        scratch += [
            pltpu.VMEM((S, 16), jnp.float32),
            pltpu.VMEM((ZR, 16), jnp.float32),
            pltpu.VMEM_SHARED((N_PAD, 16), jnp.float32),
            pltpu.SemaphoreType.DMA,
        ]

    def body(table, srcs, rels, dsts, *rest):
        if with_deg:
            (agg_out, deg_out, idx_v, rseg_v, dst_v, rows_v,
             acc_sh, gsems, ssems, ones_v, zdeg_v, deg_sh, dsem) = rest
        else:
            (agg_out, idx_v, rseg_v, dst_v, rows_v,
             acc_sh, gsems, ssems) = rest
        cid = lax.axis_index("c")
        sid = lax.axis_index("s")
        wid = sid * NC + cid
        base = sid * RPT

        pltpu.sync_copy(srcs.at[wid], idx_v)
        pltpu.sync_copy(dsts.at[wid], dst_v)

        for sg in range(CH // SEG):
            pltpu.sync_copy(rels.at[wid, pl.ds(sg * SEG, SEG)], rseg_v)

            def idx_fill(c, _):
                for j in range(S // 16):
                    sl = pl.ds(j * 16, 16)
                    r16 = rseg_v[c, sl]
                    idx_v[sg * SEG + c, sl] = (
                        lax.shift_right_logical(r16, 1) * (2 * N)
                        + idx_v[sg * SEG + c, sl] * 2
                        + lax.bitwise_and(r16, 1))
                return 0

            lax.fori_loop(0, SEG, idx_fill, 0)

        z16 = jnp.zeros((16,), jnp.float32)

        def zfill(i, _):
            for j in range(H // 16):
                rows_v[i, pl.ds(j * 16, 16)] = z16
            if with_deg:
                zdeg_v[i, :] = z16
            return 0

        lax.fori_loop(0, ZR, zfill, 0)
        for k in range(RPT // ZR):
            pltpu.sync_copy(rows_v.at[pl.ds(0, ZR)],
                            acc_sh.at[pl.ds(base + k * ZR, ZR)])
            if with_deg:
                pltpu.sync_copy(zdeg_v, deg_sh.at[pl.ds(base + k * ZR, ZR)])

        if with_deg:
            o16 = jnp.ones((16,), jnp.float32)

            def ofill(i, _):
                ones_v[i, :] = o16
                return 0

            lax.fori_loop(0, S, ofill, 0)

        plsc.subcore_barrier()

        def drain_rows(half):
            for b in range(GRP):
                pltpu.make_async_copy(
                    table.at[pl.ds(0, S)],
                    rows_v.at[pl.ds((half * GRP + b) * S, S)],
                    ssems[half]).wait()

        def drain_deg(count):
            for _ in range(count):
                pltpu.make_async_copy(
                    table.at[pl.ds(0, S), pl.ds(0, 16)], ones_v, dsem).wait()

        def fire_group(g, half):
            c0 = g * GRP
            off = half * GRP * S
            gets = [
                pltpu.async_copy(table.at[idx_v.at[c0 + b]],
                                 rows_v.at[pl.ds(off + b * S, S)],
                                 gsems[half * GRP + b])
                for b in range(GRP)
            ]
            for b in range(GRP):
                gets[b].wait()
                pltpu.async_copy(rows_v.at[pl.ds(off + b * S, S)],
                                 acc_sh.at[dst_v.at[c0 + b]],
                                 ssems[half], add=True)
                if with_deg:
                    pltpu.async_copy(ones_v, deg_sh.at[dst_v.at[c0 + b]],
                                     dsem, add=True)

        def pair(p, _):
            @pl.when(p > 0)
            def _():
                drain_rows(0)
                if with_deg:
                    drain_deg(2 * GRP)
            fire_group(2 * p, 0)

            @pl.when(p > 0)
            def _():
                drain_rows(1)
            fire_group(2 * p + 1, 1)
            return 0

        npair = (CH // GRP) // 2
        lax.fori_loop(0, npair, pair, 0)
        drain_rows(0)
        fire_group(CH // GRP - 1, 0)
        drain_rows(0)
        drain_rows(1)
        if with_deg:
            drain_deg((CH // GRP - (npair - 1) * 2) * GRP)

        plsc.subcore_barrier()

        pltpu.sync_copy(acc_sh.at[pl.ds(base, RPT)],
                        agg_out.at[cid, pl.ds(base, RPT)])
        if with_deg:
            pltpu.sync_copy(deg_sh.at[pl.ds(base, RPT)],
                            deg_out.at[cid, pl.ds(base, RPT)])

    return functools.partial(
        pl.kernel, mesh=mesh,
        out_type=tuple(out_type) if with_deg else out_type[0],
        scratch_types=scratch,
        compiler_params=pltpu.CompilerParams(use_tc_tiling_on_sc=False),
    )(body)


_sc_cache = {}


def _sc_agg_kernel(with_deg):
    if with_deg not in _sc_cache:
        _sc_cache[with_deg] = _make_sc_agg(with_deg)
    return _sc_cache[with_deg]


def kernel(node_features, edge_triples, num_nodes, W_in, b_in, basis0, att0,
           rootW0, rootb0, ln_s0, ln_b0, basis1, att1, rootW1, rootb1,
           ln_s1, ln_b1):
    src = edge_triples[:, 0].astype(jnp.int32).reshape(NW, CH, S)
    rel = edge_triples[:, 1].astype(jnp.int32).reshape(NW, CH, S)
    dst = edge_triples[:, 2].astype(jnp.int32).reshape(NW, CH, S)
    Wcat0 = jnp.einsum('rb,bio->iro', att0, basis0).reshape(H, R * H)
    Wcat1 = jnp.einsum('rb,bio->iro', att1, basis1).reshape(H, R * H)

    x, z0 = _encode(node_features, W_in, b_in.reshape(1, H), Wcat0)
    agg0, deg = _sc_agg_kernel(True)(z0.reshape(N * R, H), src, rel, dst)
    h, z1 = _mid(agg0[0], agg0[1], deg[0], deg[1], x, rootW0,
                 rootb0.reshape(1, H), ln_s0.reshape(1, H),
                 ln_b0.reshape(1, H), Wcat1)
    agg1 = _sc_agg_kernel(False)(z1.reshape(N * R, H), src, rel, dst)
    out = _final(agg1[0], agg1[1], deg[0], deg[1], h, rootW1,
                 rootb1.reshape(1, H), ln_s1.reshape(1, H),
                 ln_b1.reshape(1, H))
    return out
